# baseline (speedup 1.0000x reference)
import functools

import jax
import jax.numpy as jnp
from jax import lax
from jax.experimental import pallas as pl
from jax.experimental.pallas import tpu as pltpu
from jax.experimental.pallas import tpu_sc as plsc

VOCAB = 1_000_000
EMB = 64
LANE = 128
K = 4

_info = plsc.get_sparse_core_info()
NC, NS = _info.num_cores, _info.num_subcores
NW = NC * NS


def _build(n_rows: int):
    rows_per_w = n_rows // NW
    n_groups = rows_per_w // K

    mesh = plsc.VectorSubcoreMesh(core_axis_name="c", subcore_axis_name="s")

    @functools.partial(
        pl.kernel,
        out_type=jax.ShapeDtypeStruct((n_rows, LANE, EMB), jnp.float32),
        mesh=mesh,
        scratch_types=[
            pltpu.VMEM((K, LANE), jnp.int32),
            pltpu.VMEM((K, LANE, EMB), jnp.float32),
            pltpu.SemaphoreType.DMA,
        ],
        compiler_params=pltpu.CompilerParams(use_tc_tiling_on_sc=False),
    )
    def emb(idx_hbm, table_hbm, out_hbm, idx_v, rows_v, sem):
        wid = lax.axis_index("s") * NC + lax.axis_index("c")
        base = wid * rows_per_w

        def group(g, carry):
            row0 = base + g * K
            pltpu.sync_copy(idx_hbm.at[pl.ds(row0, K)], idx_v)
            copies = [
                pltpu.async_copy(table_hbm.at[idx_v.at[j]], rows_v.at[j], sem)
                for j in range(K)
            ]
            for c in copies:
                c.wait()
            pltpu.sync_copy(rows_v, out_hbm.at[pl.ds(row0, K)])
            return carry

        lax.fori_loop(0, n_groups, group, 0)

    return emb


def kernel(token, table):
    b, s = token.shape
    n_rows = (b * s) // LANE
    idx = token.reshape(n_rows, LANE).astype(jnp.int32)
    out = _build(n_rows)(idx, table)
    return out.reshape(b, s, EMB)


# speedup vs baseline: 1.0478x; 1.0478x over previous
import functools

import jax
import jax.numpy as jnp
from jax import lax
from jax.experimental import pallas as pl
from jax.experimental.pallas import tpu as pltpu
from jax.experimental.pallas import tpu_sc as plsc

VOCAB = 1_000_000
EMB = 64
LANE = 128
K = 4
NBUF = 2

_info = plsc.get_sparse_core_info()
NC, NS = _info.num_cores, _info.num_subcores
NW = NC * NS


def _build(n_rows: int):
    rows_per_w = n_rows // NW
    n_groups = rows_per_w // K
    assert n_groups % NBUF == 0 and n_groups >= 2 * NBUF

    mesh = plsc.VectorSubcoreMesh(core_axis_name="c", subcore_axis_name="s")

    @functools.partial(
        pl.kernel,
        out_type=jax.ShapeDtypeStruct((n_rows, LANE, EMB), jnp.float32),
        mesh=mesh,
        scratch_types=[
            pltpu.VMEM((NBUF, K, LANE), jnp.int32),
            pltpu.VMEM((NBUF, K, LANE, EMB), jnp.float32),
            [pltpu.SemaphoreType.DMA] * NBUF,
            [pltpu.SemaphoreType.DMA] * NBUF,
        ],
        compiler_params=pltpu.CompilerParams(use_tc_tiling_on_sc=False),
    )
    def emb(idx_hbm, table_hbm, out_hbm, idx_v, rows_v, gsems, wsems):
        wid = lax.axis_index("s") * NC + lax.axis_index("c")
        base = wid * rows_per_w

        def load_idx(g, b):
            pltpu.sync_copy(idx_hbm.at[pl.ds(base + g * K, K)], idx_v.at[b])

        def fire_gather(b):
            for j in range(K):
                pltpu.async_copy(
                    table_hbm.at[idx_v.at[b, j]], rows_v.at[b, j], gsems[b]
                )

        def wait_gather(b):
            for j in range(K):
                pltpu.make_async_copy(
                    table_hbm.at[idx_v.at[b, j]], rows_v.at[b, j], gsems[b]
                ).wait()

        def fire_wb(g, b):
            pltpu.async_copy(
                rows_v.at[b], out_hbm.at[pl.ds(base + g * K, K)], wsems[b]
            )

        def wait_wb(g, b):
            pltpu.make_async_copy(
                rows_v.at[b], out_hbm.at[pl.ds(base + g * K, K)], wsems[b]
            ).wait()

        for b in range(NBUF):
            load_idx(b, b)
            fire_gather(b)

        def step(t, carry):
            for b in range(NBUF):
                g = NBUF + t * NBUF + b
                wait_gather(b)
                fire_wb(g - NBUF, b)
                load_idx(g, b)
                wait_wb(g - NBUF, b)
                fire_gather(b)
            return carry

        lax.fori_loop(0, (n_groups - NBUF) // NBUF, step, 0)

        for b in range(NBUF):
            g = n_groups - NBUF + b
            wait_gather(b)
            fire_wb(g, b)
        for b in range(NBUF):
            g = n_groups - NBUF + b
            wait_wb(g, b)

    return emb


def kernel(token, table):
    b, s = token.shape
    n_rows = (b * s) // LANE
    idx = token.reshape(n_rows, LANE).astype(jnp.int32)
    out = _build(n_rows)(idx, table)
    return out.reshape(b, s, EMB)
